# baseline (speedup 1.0000x reference)
import functools

import jax
import jax.numpy as jnp
from jax import lax
from jax.experimental import pallas as pl
from jax.experimental.pallas import tpu as pltpu
from jax.experimental.pallas import tpu_sc as plsc

NUM_EMB = 1000000
DIM = 32
BATCH = 16384
SEQ = 50
QROWS = NUM_EMB // 4
QW = 128

NUM_WORKERS = 32
IBLK = BATCH // NUM_WORKERS
NSUB = IBLK // QW
NBUF = 4


def _make_lookup():
    mesh = plsc.VectorSubcoreMesh(core_axis_name="c", subcore_axis_name="s")

    @functools.partial(
        pl.kernel,
        out_type=jax.ShapeDtypeStruct((SEQ, DIM, BATCH), jnp.float32),
        mesh=mesh,
        scratch_types=[
            pltpu.VMEM((SEQ, IBLK), jnp.int32),
            [pltpu.VMEM((QW,), jnp.int32) for _ in range(NBUF)],
            [pltpu.VMEM((QW, QW), jnp.float32) for _ in range(NBUF)],
            [pltpu.VMEM((DIM, QW), jnp.float32) for _ in range(NBUF)],
            [pltpu.SemaphoreType.DMA for _ in range(NBUF)],
            [pltpu.SemaphoreType.DMA for _ in range(NBUF)],
        ],
        compiler_params=pltpu.CompilerParams(
            use_tc_tiling_on_sc=True, needs_layout_passes=False
        ),
    )
    def lookup(table_hbm, idxt_hbm, out_hbm, idxT, qbuf, rowbuf, colbuf, gsem, ssem):
        wid = lax.axis_index("s") * 2 + lax.axis_index("c")
        i0 = wid * IBLK
        pltpu.sync_copy(idxt_hbm.at[:, pl.ds(i0, IBLK)], idxT)

        lanes = lax.iota(jnp.int32, 16)
        rowiv = [g * 16 + lanes for g in range(QW // 16)]

        def qprep(j, b):
            for g in range(QW // 16):
                iv = idxT[j, pl.ds(b * QW + g * 16, 16)]
                qbuf[b][pl.ds(g * 16, 16)] = jnp.right_shift(iv, 2)

        def start_gather(b):
            pltpu.async_copy(table_hbm.at[qbuf[b]], rowbuf[b], gsem[b])

        def extract(j, b):
            subs = [
                jnp.bitwise_and(idxT[j, pl.ds(b * QW + g * 16, 16)], 3) * DIM
                for g in range(QW // 16)
            ]

            @pl.loop(0, DIM)
            def _d(d):
                for g in range(QW // 16):
                    colbuf[b][d, pl.ds(g * 16, 16)] = plsc.load_gather(
                        rowbuf[b], [rowiv[g], subs[g] + d]
                    )

        def start_store(j, b):
            pltpu.async_copy(
                colbuf[b], out_hbm.at[j, :, pl.ds(i0 + b * QW, QW)], ssem[b]
            )

        def wait_gather(b):
            pltpu.make_async_copy(
                table_hbm.at[pl.ds(0, QW), :], rowbuf[b], gsem[b]
            ).wait()

        def wait_store(b):
            pltpu.make_async_copy(
                out_hbm.at[0, :, pl.ds(0, QW)], colbuf[b], ssem[b]
            ).wait()

        for b in range(NBUF):
            qprep(0, b)
            start_gather(b)

        @pl.loop(0, SEQ)
        def _ring(g):
            for b in range(NBUF):
                wait_gather(b)

                @pl.when(g >= 1)
                def _():
                    wait_store(b)

                extract(g, b)
                start_store(g, b)

                @pl.when(g < SEQ - 1)
                def _():
                    qprep(g + 1, b)
                    start_gather(b)

        for b in range(NBUF):
            wait_store(b)

    return lookup


_lookup = _make_lookup()


def kernel(weight, indices):
    table4 = weight.reshape(QROWS, QW)
    out_t = _lookup(table4, indices.T.astype(jnp.int32))
    return jnp.transpose(out_t, (2, 0, 1))
